# baseline (speedup 1.0000x reference)
import jax
import jax.numpy as jnp
from jax.experimental import pallas as pl

_CBLK = 32


def _bcast_body(a_ref, b_ref, out_ref):
    plane = a_ref[...][:, :, None] + b_ref[...][:, None, :]
    out_ref[...] = plane[None]


def kernel(mask, row_embed, col_embed):
    batch = mask.shape[0]
    h, w = mask.shape[-2], mask.shape[-1]
    nf = row_embed.shape[1]
    c_total = 2 * nf
    zeros = jnp.zeros((nf, h), dtype=row_embed.dtype)
    a_tab = jnp.concatenate([zeros, row_embed.T], axis=0)
    b_tab = jnp.concatenate([col_embed.T, zeros], axis=0)

    grid = (batch, c_total // _CBLK)
    out = pl.pallas_call(
        _bcast_body,
        grid=grid,
        in_specs=[
            pl.BlockSpec((_CBLK, h), lambda b, j: (j, 0)),
            pl.BlockSpec((_CBLK, w), lambda b, j: (j, 0)),
        ],
        out_specs=pl.BlockSpec((1, _CBLK, h, w), lambda b, j: (b, j, 0, 0)),
        out_shape=jax.ShapeDtypeStruct((batch, c_total, h, w), row_embed.dtype),
    )(a_tab, b_tab)
    return out


# speedup vs baseline: 1.0056x; 1.0056x over previous
import jax
import jax.numpy as jnp
from jax.experimental import pallas as pl
from jax.experimental.pallas import tpu as pltpu

_CBLK = 16
_NBUF = 2


def _bcast_body(a_ref, b_ref, out_ref, scratch, sem):
    nj = pl.num_programs(0)
    j = pl.program_id(0)
    slot = jax.lax.rem(j, _NBUF)
    nb = out_ref.shape[0]

    def copies(s, jj):
        return [
            pltpu.make_async_copy(
                scratch.at[s],
                out_ref.at[b, pl.ds(jj * _CBLK, _CBLK)],
                sem.at[s, b],
            )
            for b in range(nb)
        ]

    @pl.when(j >= _NBUF)
    def _():
        for c in copies(slot, j - _NBUF):
            c.wait()

    scratch[slot] = a_ref[...][:, :, None] + b_ref[...][:, None, :]

    for c in copies(slot, j):
        c.start()

    @pl.when(j == nj - 1)
    def _():
        for d in range(_NBUF - 1, -1, -1):
            jj = j - d
            s = jax.lax.rem(jj, _NBUF)
            for c in copies(s, jj):
                c.wait()


def kernel(mask, row_embed, col_embed):
    batch = mask.shape[0]
    h, w = mask.shape[-2], mask.shape[-1]
    nf = row_embed.shape[1]
    c_total = 2 * nf
    zeros = jnp.zeros((nf, h), dtype=row_embed.dtype)
    a_tab = jnp.concatenate([zeros, row_embed.T], axis=0)
    b_tab = jnp.concatenate([col_embed.T, zeros], axis=0)

    grid = (c_total // _CBLK,)
    out = pl.pallas_call(
        _bcast_body,
        grid=grid,
        in_specs=[
            pl.BlockSpec((_CBLK, h), lambda j: (j, 0)),
            pl.BlockSpec((_CBLK, w), lambda j: (j, 0)),
        ],
        out_specs=pl.BlockSpec(memory_space=pltpu.MemorySpace.HBM),
        out_shape=jax.ShapeDtypeStruct((batch, c_total, h, w), row_embed.dtype),
        scratch_shapes=[
            pltpu.VMEM((_NBUF, _CBLK, h, w), row_embed.dtype),
            pltpu.SemaphoreType.DMA((_NBUF, batch)),
        ],
    )(a_tab, b_tab)
    return out


# speedup vs baseline: 4.9290x; 4.9017x over previous
import jax
import jax.numpy as jnp
from jax.experimental import pallas as pl

_YBLK = 40


def _bcast_body(row_ref, col_ref, out_ref):
    nf = row_ref.shape[1]
    re = row_ref[...]
    ce = col_ref[...]
    yb, w = out_ref.shape[1], out_ref.shape[2]
    out_ref[0, :, :, nf:] = jnp.broadcast_to(re[:, None, :], (yb, w, nf))
    out_ref[0, :, :, :nf] = jnp.broadcast_to(ce[None, :, :], (yb, w, nf))


def kernel(mask, row_embed, col_embed):
    batch = mask.shape[0]
    h, w = mask.shape[-2], mask.shape[-1]
    nf = row_embed.shape[1]

    grid = (batch, h // _YBLK)
    p = pl.pallas_call(
        _bcast_body,
        grid=grid,
        in_specs=[
            pl.BlockSpec((_YBLK, nf), lambda b, i: (i, 0)),
            pl.BlockSpec((w, nf), lambda b, i: (0, 0)),
        ],
        out_specs=pl.BlockSpec((1, _YBLK, w, 2 * nf), lambda b, i: (b, i, 0, 0)),
        out_shape=jax.ShapeDtypeStruct((batch, h, w, 2 * nf), row_embed.dtype),
    )(row_embed, col_embed)
    return jnp.transpose(p, (0, 3, 1, 2))
